# baseline (speedup 1.0000x reference)
import jax
import jax.numpy as jnp
from jax import lax
from jax.experimental import pallas as pl
from jax.experimental.pallas import tpu as pltpu
from jax.experimental.pallas import tpu_sc as plsc

BATCH = 16384
DIM = 16
ROWS_PER_SLICE = 8

_info = plsc.get_sparse_core_info()
_NC, _NS = _info.num_cores, _info.num_subcores
_NW = _NC * _NS
_B_PER_W = BATCH // _NW
_CHUNK = 128
_N_CHUNKS = _B_PER_W // _CHUNK


def _gmf_body(uid_hbm, iid_hbm, utab_hbm, itab_hbm, out_hbm,
              uidx_v, iidx_v, ugidx_v, igidx_v, ubuf, ibuf, out_v, sem):
    wid = lax.axis_index("s") * _NC + lax.axis_index("c")
    base = wid * _B_PER_W
    pltpu.sync_copy(uid_hbm.at[pl.ds(base, _B_PER_W)], uidx_v)
    pltpu.sync_copy(iid_hbm.at[pl.ds(base, _B_PER_W)], iidx_v)

    iota = lax.iota(jnp.int32, 16)

    for g in range(_B_PER_W // 16):
        u = uidx_v[pl.ds(g * 16, 16)]
        i = iidx_v[pl.ds(g * 16, 16)]
        c = g // 8
        o = (g % 8) * 16
        ugidx_v[c, pl.ds(o, 16)] = u >> 3
        igidx_v[c, pl.ds(o, 16)] = i >> 3

    for c in range(_N_CHUNKS):
        cu = pltpu.async_copy(utab_hbm.at[ugidx_v.at[c]], ubuf, sem)
        ci = pltpu.async_copy(itab_hbm.at[igidx_v.at[c]], ibuf, sem)
        cu.wait()
        ci.wait()

        def extract_body(g, carry, c=c):
            iota = lax.iota(jnp.int32, 16)
            j0 = c * _CHUNK + g * 16
            uvec = uidx_v[pl.ds(j0, 16)]
            ivec = iidx_v[pl.ds(j0, 16)]
            rows = g * 16 + iota
            pu = (uvec & 7) << 4
            pi = (ivec & 7) << 4
            o0 = (j0 // 8) + (iota >> 3)
            o1b = (iota & 7) << 4
            for k in range(DIM):
                uv = plsc.load_gather(ubuf, [rows, pu + k])
                iv = plsc.load_gather(ibuf, [rows, pi + k])
                plsc.store_scatter(out_v, [o0, o1b + k], uv * iv)
            return carry

        lax.fori_loop(0, _CHUNK // 16, extract_body, 0)

    pltpu.sync_copy(out_v, out_hbm.at[pl.ds(wid * (_B_PER_W // 8), _B_PER_W // 8)])


@jax.jit
def kernel(user_id, item_id, user_embed, item_embed):
    f = pl.kernel(
        _gmf_body,
        out_type=jax.ShapeDtypeStruct((BATCH * DIM // 128, 128), jnp.float32),
        mesh=plsc.VectorSubcoreMesh(core_axis_name="c", subcore_axis_name="s"),
        compiler_params=pltpu.CompilerParams(needs_layout_passes=False),
        scratch_types=[
            pltpu.VMEM((_B_PER_W,), jnp.int32),
            pltpu.VMEM((_B_PER_W,), jnp.int32),
            pltpu.VMEM((_N_CHUNKS, _CHUNK), jnp.int32),
            pltpu.VMEM((_N_CHUNKS, _CHUNK), jnp.int32),
            pltpu.VMEM((_CHUNK, 128), jnp.float32),
            pltpu.VMEM((_CHUNK, 128), jnp.float32),
            pltpu.VMEM((_B_PER_W // 8, 128), jnp.float32),
            pltpu.SemaphoreType.DMA,
        ],
    )
    utab = user_embed.reshape(-1, 128)
    itab = item_embed.reshape(-1, 128)
    out = f(user_id.astype(jnp.int32), item_id.astype(jnp.int32), utab, itab)
    return out.reshape(BATCH, DIM)


# speedup vs baseline: 5.4859x; 5.4859x over previous
import jax
import jax.numpy as jnp
from jax import lax
from jax.experimental import pallas as pl
from jax.experimental.pallas import tpu as pltpu
from jax.experimental.pallas import tpu_sc as plsc

BATCH = 16384
DIM = 16

_info = plsc.get_sparse_core_info()
_NC, _NS = _info.num_cores, _info.num_subcores
_NW = _NC * _NS
_B_PER_W = BATCH // _NW
_CHUNK = 16
_N_CHUNKS = _B_PER_W // _CHUNK


def _gmf_body(uid_hbm, iid_hbm, utab_hbm, itab_hbm, out_hbm,
              uidx_v, iidx_v, ubuf, ibuf, out_v, sem_u, sem_i):
    wid = lax.axis_index("s") * _NC + lax.axis_index("c")
    base = wid * _B_PER_W

    pltpu.sync_copy(uid_hbm.at[pl.ds(base, _B_PER_W)], uidx_v)
    pltpu.sync_copy(iid_hbm.at[pl.ds(base, _B_PER_W)], iidx_v)

    def chunk_body(c, carry):
        iota = lax.iota(jnp.int32, 16)
        uvec = uidx_v[pl.ds(c * _CHUNK, _CHUNK)]
        ivec = iidx_v[pl.ds(c * _CHUNK, _CHUNK)]
        u0v = (uvec >> 7) << 7
        i0v = (ivec >> 7) << 7
        ulane = uvec & 127
        ilane = ivec & 127
        for j in range(_CHUNK):
            u0 = pl.multiple_of(u0v[j], 128)
            i0 = pl.multiple_of(i0v[j], 128)
            pltpu.async_copy(utab_hbm.at[:, pl.ds(u0, 128)], ubuf.at[j], sem_u)
            pltpu.async_copy(itab_hbm.at[:, pl.ds(i0, 128)], ibuf.at[j], sem_i)
        for j in range(_CHUNK):
            pltpu.make_async_copy(
                utab_hbm.at[:, pl.ds(0, 128)], ubuf.at[j], sem_u).wait()
            pltpu.make_async_copy(
                itab_hbm.at[:, pl.ds(0, 128)], ibuf.at[j], sem_i).wait()
        for j in range(_CHUNK):
            jv = jnp.full((16,), j, jnp.int32)
            ug = plsc.load_gather(
                ubuf, [jv, iota, jnp.full((16,), ulane[j], jnp.int32)])
            ig = plsc.load_gather(
                ibuf, [jv, iota, jnp.full((16,), ilane[j], jnp.int32)])
            plsc.store_scatter(
                out_v, [iota, jnp.full((16,), c * _CHUNK + j, jnp.int32)],
                ug * ig)
        return carry

    lax.fori_loop(0, _N_CHUNKS, chunk_body, 0)

    for k in range(DIM):
        pltpu.sync_copy(out_v.at[k], out_hbm.at[k, pl.ds(base, _B_PER_W)])


@jax.jit
def kernel(user_id, item_id, user_embed, item_embed):
    f = pl.kernel(
        _gmf_body,
        out_type=jax.ShapeDtypeStruct((DIM, BATCH), jnp.float32),
        mesh=plsc.VectorSubcoreMesh(core_axis_name="c", subcore_axis_name="s"),
        compiler_params=pltpu.CompilerParams(needs_layout_passes=False),
        scratch_types=[
            pltpu.VMEM((_B_PER_W,), jnp.int32),
            pltpu.VMEM((_B_PER_W,), jnp.int32),
            pltpu.VMEM((_CHUNK, DIM, 128), jnp.float32),
            pltpu.VMEM((_CHUNK, DIM, 128), jnp.float32),
            pltpu.VMEM((DIM, _B_PER_W), jnp.float32),
            pltpu.SemaphoreType.DMA,
            pltpu.SemaphoreType.DMA,
        ],
    )
    out_t = f(user_id.astype(jnp.int32), item_id.astype(jnp.int32),
              user_embed.T, item_embed.T)
    return out_t.T
